# speedup vs baseline: 1.1646x; 1.1646x over previous
import jax
import jax.numpy as jnp
from jax.experimental import pallas as pl

N_NODES = 10000
HEADS1 = 8
HID = 16
OUT = 16


def _gat_layer(x, src, dst, W, att_src, att_dst, bias, heads, out_ch):
    N = x.shape[0]
    h = (x @ W).reshape(N, heads, out_ch)
    a_src = (h * att_src).sum(axis=-1)
    a_dst = (h * att_dst).sum(axis=-1)
    e = a_src[src] + a_dst[dst]
    e = jax.nn.leaky_relu(e, negative_slope=0.2)
    C = jnp.max(e, axis=0)
    ex = jnp.exp(e - C)
    denom = jax.ops.segment_sum(ex, dst, num_segments=N)
    msg = h[src] * ex[:, :, None]
    out = jax.ops.segment_sum(msg, dst, num_segments=N)
    out = out / (denom[:, :, None] + 1e-16)
    return out.reshape(N, heads * out_ch) + bias


def _logsoftmax_kernel(x_ref, o_ref):
    x = x_ref[...]
    m = jnp.max(x, axis=1, keepdims=True)
    s = jnp.log(jnp.sum(jnp.exp(x - m), axis=1, keepdims=True))
    o_ref[...] = x - m - s


def kernel(x, edge_index, W1, a_src1, a_dst1, b1, W2, a_src2, a_dst2, b2):
    N = x.shape[0]
    loop = jnp.arange(N, dtype=jnp.int32)
    src = jnp.concatenate([edge_index[0].astype(jnp.int32), loop])
    dst = jnp.concatenate([edge_index[1].astype(jnp.int32), loop])
    h = _gat_layer(x, src, dst, W1, a_src1, a_dst1, b1, HEADS1, HID)
    h = jax.nn.elu(h)
    h = _gat_layer(h, src, dst, W2, a_src2, a_dst2, b2, 1, OUT)
    return pl.pallas_call(
        _logsoftmax_kernel,
        out_shape=jax.ShapeDtypeStruct((N, OUT), jnp.float32),
        grid=(10,),
        in_specs=[pl.BlockSpec((N // 10, OUT), lambda i: (i, 0))],
        out_specs=pl.BlockSpec((N // 10, OUT), lambda i: (i, 0)),
    )(h)


# speedup vs baseline: 51.9344x; 44.5923x over previous
import functools

import jax
import jax.numpy as jnp
from jax import lax
from jax.experimental import pallas as pl
from jax.experimental.pallas import tpu as pltpu
from jax.experimental.pallas import tpu_sc as plsc

N = 10000
NPAD = 10240
E_REAL = 330000
NW = 32
EPT = 10368
EPAD = NW * EPT
CH = 81
K = 128
THIRD = EPT // 3
ROWS_PT = NPAD // 16

_MESH = dict(core_axis_name="c", subcore_axis_name="s", num_cores=2,
             num_subcores=16)


def _tc_prep1(x_ref, w_ref, asr_ref, adr_ref, g_ref, ast_ref, adt_ref):
    h = jnp.dot(x_ref[...], w_ref[...], preferred_element_type=jnp.float32)
    g_ref[...] = h
    sel = (lax.broadcasted_iota(jnp.int32, (128, 8), 0) // 16
           == lax.broadcasted_iota(jnp.int32, (128, 8), 1)).astype(jnp.float32)
    asr = jnp.dot(h * asr_ref[...], sel, preferred_element_type=jnp.float32)
    adr = jnp.dot(h * adr_ref[...], sel, preferred_element_type=jnp.float32)
    ast_ref[...] = asr.T
    adt_ref[...] = adr.T


def _tc_mid(an_ref, ad_ref, b1_ref, w2_ref, s2_ref, d2_ref, g2_ref, a2_ref):
    num = an_ref[0] + an_ref[1]
    den = ad_ref[0] + ad_ref[1]
    den128 = jnp.repeat(den[:, 0:8], 16, axis=1)
    o = num / (den128 + 1e-16) + b1_ref[...][None, :]
    h2 = jnp.where(o > 0, o, jnp.exp(o) - 1.0)
    g = jnp.dot(h2, w2_ref[...], preferred_element_type=jnp.float32)
    g2_ref[...] = g
    s2 = jnp.dot(g, s2_ref[...].T, preferred_element_type=jnp.float32)
    d2 = jnp.dot(g, d2_ref[...].T, preferred_element_type=jnp.float32)
    B = s2.shape[0]
    a2_ref[...] = jnp.concatenate(
        [s2.T, d2.T, jnp.zeros((6, B), jnp.float32)], axis=0)


def _tc_final(an_ref, ad_ref, b2_ref, o_ref):
    num = an_ref[0] + an_ref[1]
    den = ad_ref[0] + ad_ref[1]
    o = num / (den[:, 0:1] + 1e-16) + b2_ref[...][None, :]
    m = jnp.max(o, axis=1, keepdims=True)
    s = jnp.log(jnp.sum(jnp.exp(o - m), axis=1, keepdims=True))
    o_ref[...] = o - m - s


def _sc_logits1(ast_hbm, adt_hbm, src_hbm, dst_hbm, eb_hbm, pmax_hbm,
                tsrc, tdst, tasrc, tadst, ebuf, maxbuf):
    cid = lax.axis_index("c")
    sid = lax.axis_index("s")
    w = sid * 2 + cid
    pltpu.sync_copy(src_hbm.at[pl.ds(w * EPT, EPT)], tsrc)
    pltpu.sync_copy(dst_hbm.at[pl.ds(w * EPT, EPT)], tdst)
    iota = lax.iota(jnp.int32, 16)
    neg = jnp.full((16,), -3.0e38, jnp.float32)
    for h in range(8):
        maxbuf[h] = neg
    for t in range(3):
        for h in range(8):
            pltpu.sync_copy(ast_hbm.at[pl.ds(h * NPAD, NPAD)], tasrc)
            pltpu.sync_copy(adt_hbm.at[pl.ds(h * NPAD, NPAD)], tadst)
            col_a = jnp.full((16,), h, jnp.int32)
            col_b = jnp.full((16,), h + 8, jnp.int32)

            def body(j, mx, t=t, col_a=col_a, col_b=col_b):
                o = t * THIRD + j * 16
                sv = tsrc[pl.ds(o, 16)]
                dv = tdst[pl.ds(o, 16)]
                e = (plsc.load_gather(tasrc, [sv])
                     + plsc.load_gather(tadst, [dv]))
                e = jnp.maximum(e, 0.2 * e)
                iv = iota + j * 16
                plsc.store_scatter(ebuf, [iv, col_a], e)
                plsc.store_scatter(ebuf, [iv, col_b], e)
                return jnp.maximum(mx, e)

            maxbuf[h] = lax.fori_loop(0, THIRD // 16, body, maxbuf[h])
        pltpu.sync_copy(ebuf, eb_hbm.at[w, t])
    pltpu.sync_copy(maxbuf, pmax_hbm.at[w])


def _sc_agg1(g1_hbm, eb_hbm, src_hbm, dst3_hbm, pmax_hbm, accn_hbm, accd_hbm,
             hbuf, ebuf, tdst, sidx, pall, accN, accD, gsem):
    cid = lax.axis_index("c")
    sid = lax.axis_index("s")
    w = sid * 2 + cid
    pltpu.sync_copy(dst3_hbm.at[w], tdst)
    pltpu.sync_copy(pmax_hbm, pall)
    neg = jnp.full((16,), -3.0e38, jnp.float32)
    iota = lax.iota(jnp.int32, 16)
    CC = jnp.zeros((16,), jnp.float32)
    for h in range(8):
        def mbody(w2, mx, h=h):
            return jnp.maximum(mx, pall[w2, h])
        mxv = lax.fori_loop(0, NW, mbody, neg)
        c_h = jnp.max(mxv)
        CC = jnp.where((iota & 7) == h, c_h, CC)
    zero16 = jnp.zeros((16,), jnp.float32)

    def zb(k, _):
        for h in range(8):
            hbuf[k, pl.ds(16 * h, 16)] = zero16
        ebuf[k] = zero16
        return 0

    lax.fori_loop(0, K, zb, 0)
    rows0 = sid * ROWS_PT
    for r in range(ROWS_PT // K):
        pltpu.sync_copy(hbuf, accN.at[pl.ds(rows0 + r * K, K), :])
        pltpu.sync_copy(ebuf, accD.at[pl.ds(rows0 + r * K, K), :])
    plsc.subcore_barrier()

    def chunk(c, _):
        pltpu.sync_copy(src_hbm.at[pl.ds(w * EPT + c * K, K)], sidx)
        pltpu.sync_copy(eb_hbm.at[w, c], ebuf)
        pltpu.async_copy(g1_hbm.at[sidx], hbuf, gsem).wait()

        def sc_(k, _2):
            ev = jnp.exp(ebuf[k] - CC)
            ebuf[k] = ev
            for h in range(8):
                s = ev[h]
                hbuf[k, pl.ds(16 * h, 16)] = hbuf[k, pl.ds(16 * h, 16)] * s
            return 0

        lax.fori_loop(0, K, sc_, 0)
        pltpu.sync_copy(hbuf, accN.at[tdst.at[c]], add=True)
        pltpu.sync_copy(ebuf, accD.at[tdst.at[c]], add=True)
        return 0

    lax.fori_loop(0, CH, chunk, 0)
    plsc.subcore_barrier()
    pltpu.sync_copy(accN.at[pl.ds(rows0, ROWS_PT), :],
                    accn_hbm.at[cid, pl.ds(rows0, ROWS_PT), :])
    pltpu.sync_copy(accD.at[pl.ds(rows0, ROWS_PT), :],
                    accd_hbm.at[cid, pl.ds(rows0, ROWS_PT), :])


def _sc_logits2(a2_hbm, src_hbm, dst_hbm, e2_hbm, pmax_hbm,
                tsrc, tdst, ta, tb, e2buf, pbuf):
    cid = lax.axis_index("c")
    sid = lax.axis_index("s")
    w = sid * 2 + cid
    pltpu.sync_copy(src_hbm.at[pl.ds(w * EPT, EPT)], tsrc)
    pltpu.sync_copy(dst_hbm.at[pl.ds(w * EPT, EPT)], tdst)
    pltpu.sync_copy(a2_hbm.at[pl.ds(0, NPAD)], ta)
    pltpu.sync_copy(a2_hbm.at[pl.ds(NPAD, NPAD)], tb)

    def body(j, mx):
        o = j * 16
        sv = tsrc[pl.ds(o, 16)]
        dv = tdst[pl.ds(o, 16)]
        e = plsc.load_gather(ta, [sv]) + plsc.load_gather(tb, [dv])
        e = jnp.maximum(e, 0.2 * e)
        e2buf[pl.ds(o, 16)] = e
        return jnp.maximum(mx, e)

    mx = lax.fori_loop(0, EPT // 16, body,
                       jnp.full((16,), -3.0e38, jnp.float32))
    pbuf[0] = mx
    pltpu.sync_copy(e2buf, e2_hbm.at[pl.ds(w * EPT, EPT)])
    pltpu.sync_copy(pbuf, pmax_hbm.at[w])


def _sc_agg2(g2_hbm, e2_hbm, src_hbm, dst3_hbm, pmax_hbm, accn_hbm, accd_hbm,
             gbuf, dxd, exbuf, tdst, sidx, pm, accN, accD, gsem):
    cid = lax.axis_index("c")
    sid = lax.axis_index("s")
    w = sid * 2 + cid
    pltpu.sync_copy(dst3_hbm.at[w], tdst)
    pltpu.sync_copy(pmax_hbm, pm)
    iota = lax.iota(jnp.int32, 16)
    e0mask = jnp.where(iota == 0, 1.0, 0.0).astype(jnp.float32)

    def mbody(w2, mx):
        return jnp.maximum(mx, pm[w2, 0])

    mxv = lax.fori_loop(0, NW, mbody, jnp.full((16,), -3.0e38, jnp.float32))
    c2 = jnp.max(mxv)
    zero16 = jnp.zeros((16,), jnp.float32)

    def zb(k, _):
        gbuf[k] = zero16
        dxd[k] = zero16
        return 0

    lax.fori_loop(0, K, zb, 0)
    rows0 = sid * ROWS_PT
    for r in range(ROWS_PT // K):
        pltpu.sync_copy(gbuf, accN.at[pl.ds(rows0 + r * K, K), :])
        pltpu.sync_copy(dxd, accD.at[pl.ds(rows0 + r * K, K), :])
    plsc.subcore_barrier()

    def chunk(c, _):
        pltpu.sync_copy(src_hbm.at[pl.ds(w * EPT + c * K, K)], sidx)
        pltpu.sync_copy(e2_hbm.at[pl.ds(w * EPT + c * K, K)], exbuf)
        pltpu.async_copy(g2_hbm.at[sidx], gbuf, gsem).wait()

        def sc_(j, _2):
            ev = jnp.exp(exbuf[pl.ds(j * 16, 16)] - c2)
            for l in range(16):
                s = ev[l]
                k = j * 16 + l
                gbuf[k] = gbuf[k] * s
                dxd[k] = e0mask * s
            return 0

        lax.fori_loop(0, K // 16, sc_, 0)
        pltpu.sync_copy(gbuf, accN.at[tdst.at[c]], add=True)
        pltpu.sync_copy(dxd, accD.at[tdst.at[c]], add=True)
        return 0

    lax.fori_loop(0, CH, chunk, 0)
    plsc.subcore_barrier()
    pltpu.sync_copy(accN.at[pl.ds(rows0, ROWS_PT), :],
                    accn_hbm.at[cid, pl.ds(rows0, ROWS_PT), :])
    pltpu.sync_copy(accD.at[pl.ds(rows0, ROWS_PT), :],
                    accd_hbm.at[cid, pl.ds(rows0, ROWS_PT), :])


def _make_sc_kernels():
    mesh = plsc.VectorSubcoreMesh(**_MESH)
    f32, i32 = jnp.float32, jnp.int32
    sc1 = functools.partial(
        pl.kernel, _sc_logits1,
        out_type=(jax.ShapeDtypeStruct((NW, 3, THIRD, 16), f32),
                  jax.ShapeDtypeStruct((NW, 8, 16), f32)),
        mesh=mesh,
        compiler_params=pltpu.CompilerParams(needs_layout_passes=False, use_tc_tiling_on_sc=False),
        scratch_types=[
            pltpu.VMEM((EPT,), i32), pltpu.VMEM((EPT,), i32),
            pltpu.VMEM((NPAD,), f32), pltpu.VMEM((NPAD,), f32),
            pltpu.VMEM((THIRD, 16), f32), pltpu.VMEM((8, 16), f32),
        ])()
    sc2 = functools.partial(
        pl.kernel, _sc_agg1,
        out_type=(jax.ShapeDtypeStruct((2, NPAD, 128), f32),
                  jax.ShapeDtypeStruct((2, NPAD, 16), f32)),
        mesh=mesh,
        compiler_params=pltpu.CompilerParams(needs_layout_passes=False, use_tc_tiling_on_sc=False),
        scratch_types=[
            pltpu.VMEM((K, 128), f32), pltpu.VMEM((K, 16), f32),
            pltpu.VMEM((CH, K), i32), pltpu.VMEM((K,), i32),
            pltpu.VMEM((NW, 8, 16), f32),
            pltpu.VMEM_SHARED((NPAD, 128), f32),
            pltpu.VMEM_SHARED((NPAD, 16), f32),
            pltpu.SemaphoreType.DMA,
        ])()
    sc3 = functools.partial(
        pl.kernel, _sc_logits2,
        out_type=(jax.ShapeDtypeStruct((EPAD,), f32),
                  jax.ShapeDtypeStruct((NW, 1, 16), f32)),
        mesh=mesh,
        compiler_params=pltpu.CompilerParams(needs_layout_passes=False, use_tc_tiling_on_sc=False),
        scratch_types=[
            pltpu.VMEM((EPT,), i32), pltpu.VMEM((EPT,), i32),
            pltpu.VMEM((NPAD,), f32), pltpu.VMEM((NPAD,), f32),
            pltpu.VMEM((EPT,), f32), pltpu.VMEM((1, 16), f32),
        ])()
    sc4 = functools.partial(
        pl.kernel, _sc_agg2,
        out_type=(jax.ShapeDtypeStruct((2, NPAD, 16), f32),
                  jax.ShapeDtypeStruct((2, NPAD, 16), f32)),
        mesh=mesh,
        compiler_params=pltpu.CompilerParams(needs_layout_passes=False, use_tc_tiling_on_sc=False),
        scratch_types=[
            pltpu.VMEM((K, 16), f32), pltpu.VMEM((K, 16), f32),
            pltpu.VMEM((K,), f32),
            pltpu.VMEM((CH, K), i32), pltpu.VMEM((K,), i32),
            pltpu.VMEM((NW, 1, 16), f32),
            pltpu.VMEM_SHARED((NPAD, 16), f32),
            pltpu.VMEM_SHARED((NPAD, 16), f32),
            pltpu.SemaphoreType.DMA,
        ])()
    return sc1, sc2, sc3, sc4


_SC1, _SC2, _SC3, _SC4 = _make_sc_kernels()


def kernel(x, edge_index, W1, a_src1, a_dst1, b1, W2, a_src2, a_dst2, b2):
    f32 = jnp.float32
    loop = jnp.arange(N, dtype=jnp.int32)
    pad = jnp.full((EPAD - E_REAL,), N, jnp.int32)
    src = jnp.concatenate([edge_index[0].astype(jnp.int32), loop, pad])
    dst = jnp.concatenate([edge_index[1].astype(jnp.int32), loop, pad])
    dst3 = dst.reshape(NW, CH, K)
    x_pad = jnp.zeros((NPAD, 128), f32).at[:N].set(x)

    BLK = 1024
    grid = NPAD // BLK
    g1, ast, adt = pl.pallas_call(
        _tc_prep1,
        grid=(grid,),
        in_specs=[
            pl.BlockSpec((BLK, 128), lambda i: (i, 0)),
            pl.BlockSpec((128, 128), lambda i: (0, 0)),
            pl.BlockSpec((1, 128), lambda i: (0, 0)),
            pl.BlockSpec((1, 128), lambda i: (0, 0)),
        ],
        out_specs=[
            pl.BlockSpec((BLK, 128), lambda i: (i, 0)),
            pl.BlockSpec((8, BLK), lambda i: (0, i)),
            pl.BlockSpec((8, BLK), lambda i: (0, i)),
        ],
        out_shape=[
            jax.ShapeDtypeStruct((NPAD, 128), f32),
            jax.ShapeDtypeStruct((8, NPAD), f32),
            jax.ShapeDtypeStruct((8, NPAD), f32),
        ],
    )(x_pad, W1, a_src1.reshape(1, 128), a_dst1.reshape(1, 128))

    astf = ast.reshape(8 * NPAD)
    adtf = adt.reshape(8 * NPAD)
    eb, pmax = _SC1(astf, adtf, src, dst)
    eb4 = eb.reshape(NW, CH, K, 16)
    accn, accd = _SC2(g1, eb4, src, dst3, pmax)

    g2, a2 = pl.pallas_call(
        _tc_mid,
        grid=(grid,),
        in_specs=[
            pl.BlockSpec((2, BLK, 128), lambda i: (0, i, 0)),
            pl.BlockSpec((2, BLK, 16), lambda i: (0, i, 0)),
            pl.BlockSpec((128,), lambda i: (0,)),
            pl.BlockSpec((128, 16), lambda i: (0, 0)),
            pl.BlockSpec((1, 16), lambda i: (0, 0)),
            pl.BlockSpec((1, 16), lambda i: (0, 0)),
        ],
        out_specs=[
            pl.BlockSpec((BLK, 16), lambda i: (i, 0)),
            pl.BlockSpec((8, BLK), lambda i: (0, i)),
        ],
        out_shape=[
            jax.ShapeDtypeStruct((NPAD, 16), f32),
            jax.ShapeDtypeStruct((8, NPAD), f32),
        ],
    )(accn, accd, b1, W2, a_src2, a_dst2)

    e2, pmax2 = _SC3(a2.reshape(8 * NPAD), src, dst)
    accn2, accd2 = _SC4(g2, e2, src, dst3, pmax2)

    BLK2 = 1000
    out = pl.pallas_call(
        _tc_final,
        grid=(N // BLK2,),
        in_specs=[
            pl.BlockSpec((2, BLK2, 16), lambda i: (0, i, 0)),
            pl.BlockSpec((2, BLK2, 16), lambda i: (0, i, 0)),
            pl.BlockSpec((16,), lambda i: (0,)),
        ],
        out_specs=pl.BlockSpec((BLK2, 16), lambda i: (i, 0)),
        out_shape=jax.ShapeDtypeStruct((N, 16), f32),
    )(accn2[:, :N], accd2[:, :N], b2)
    return out


# speedup vs baseline: 89.6491x; 1.7262x over previous
import functools

import jax
import jax.numpy as jnp
from jax import lax
from jax.experimental import pallas as pl
from jax.experimental.pallas import tpu as pltpu
from jax.experimental.pallas import tpu_sc as plsc

N = 10000
NPAD = 10240
E_REAL = 330000
NW = 32
EPT = 10368
EPAD = NW * EPT
CH = 108
K = 96
CHA = 324
KA = 32
ROWS_PT = NPAD // 16
NBUF = 4

_MESH = dict(core_axis_name="c", subcore_axis_name="s", num_cores=2,
             num_subcores=16)
_SC_PARAMS = pltpu.CompilerParams(needs_layout_passes=False,
                                  use_tc_tiling_on_sc=False)
_ZCHUNKS = (96, 96, 96, 96, 96, 96, 64)


def _tc_prep1(x_ref, w_ref, asr_ref, adr_ref, g_ref, as_ref, ad_ref,
              ms_ref, md_ref):
    h = jnp.dot(x_ref[...], w_ref[...], preferred_element_type=jnp.float32)
    g_ref[...] = h
    sel = (lax.broadcasted_iota(jnp.int32, (128, 8), 0) // 16
           == lax.broadcasted_iota(jnp.int32, (128, 8), 1)).astype(jnp.float32)
    asr = jnp.dot(h * asr_ref[...], sel, preferred_element_type=jnp.float32)
    adr = jnp.dot(h * adr_ref[...], sel, preferred_element_type=jnp.float32)
    as_ref[...] = jnp.concatenate([asr, asr], axis=1)
    ad_ref[...] = jnp.concatenate([adr, adr], axis=1)
    bs = jnp.max(asr, axis=0, keepdims=True)
    bd = jnp.max(adr, axis=0, keepdims=True)
    bs = jnp.concatenate([bs, bs], axis=1)
    bd = jnp.concatenate([bd, bd], axis=1)
    i = pl.program_id(0)
    big = jnp.full((1, 16), -3.0e38, jnp.float32)
    ms_ref[...] = jnp.maximum(jnp.where(i == 0, big, ms_ref[...]), bs)
    md_ref[...] = jnp.maximum(jnp.where(i == 0, big, md_ref[...]), bd)


def _tc_mid(an_ref, ad_ref, b1_ref, w2_ref, s2_ref, d2_ref,
            g2_ref, a2_ref, m2_ref):
    num = an_ref[0] + an_ref[1]
    den = ad_ref[0] + ad_ref[1]
    den128 = jnp.repeat(den[:, 0:8], 16, axis=1)
    o = num / (den128 + 1e-16) + b1_ref[...][None, :]
    h2 = jnp.where(o > 0, o, jnp.exp(o) - 1.0)
    g = jnp.dot(h2, w2_ref[...], preferred_element_type=jnp.float32)
    g2_ref[...] = g
    s2 = jnp.dot(g, s2_ref[...].T, preferred_element_type=jnp.float32)
    d2 = jnp.dot(g, d2_ref[...].T, preferred_element_type=jnp.float32)
    B = s2.shape[0]
    a2_ref[...] = jnp.concatenate(
        [s2.T, d2.T, jnp.zeros((6, B), jnp.float32)], axis=0)
    bs = jnp.broadcast_to(jnp.max(s2, axis=0, keepdims=True), (1, 8))
    bd = jnp.broadcast_to(jnp.max(d2, axis=0, keepdims=True), (1, 8))
    m2 = jnp.concatenate([bs, bd], axis=1)
    i = pl.program_id(0)
    big = jnp.full((1, 16), -3.0e38, jnp.float32)
    m2_ref[...] = jnp.maximum(jnp.where(i == 0, big, m2_ref[...]), m2)


def _tc_final(an_ref, ad_ref, b2_ref, o_ref):
    num = an_ref[0] + an_ref[1]
    den = ad_ref[0] + ad_ref[1]
    o = num / (den[:, 0:1] + 1e-16) + b2_ref[...][None, :]
    m = jnp.max(o, axis=1, keepdims=True)
    s = jnp.log(jnp.sum(jnp.exp(o - m), axis=1, keepdims=True))
    o_ref[...] = o - m - s


def _sc_agg1(g1_hbm, as_hbm, ad_hbm, src_hbm, dst_hbm, ms_hbm, md_hbm,
             accn_hbm, accd_hbm,
             tsrc, tdst, ccb,
             hb0, hb1, hb2, ab0, ab1, ab2,
             db0, db1, db2, xd0, xd1, xd2,
             accN, accD,
             gs0, gs1, gs2, ss0, ss1, ss2):
    cid = lax.axis_index("c")
    sid = lax.axis_index("s")
    w = sid * 2 + cid
    pltpu.sync_copy(src_hbm.at[w], tsrc.at[pl.ds(0, CHA), :])
    pltpu.sync_copy(dst_hbm.at[w], tdst.at[pl.ds(0, CHA), :])
    pltpu.sync_copy(src_hbm.at[w, pl.ds(0, 2), :], tsrc.at[pl.ds(CHA, 2), :])
    pltpu.sync_copy(dst_hbm.at[w, pl.ds(0, 2), :], tdst.at[pl.ds(CHA, 2), :])
    pltpu.sync_copy(ms_hbm, ccb)
    msv = ccb[0]
    pltpu.sync_copy(md_hbm, ccb)
    t = msv + ccb[0]
    CC = jnp.maximum(t, 0.2 * t)

    hbufs = (hb0, hb1, hb2)
    abufs = (ab0, ab1, ab2)
    dbufs = (db0, db1, db2)
    xbufs = (xd0, xd1, xd2)
    gsems = (gs0, gs1, gs2)
    ssems = (ss0, ss1, ss2)

    zero16 = jnp.zeros((16,), jnp.float32)

    def zb(k, _):
        for hh in range(8):
            hb0[k, pl.ds(16 * hh, 16)] = zero16
            hb2[k, pl.ds(16 * hh, 16)] = zero16
        xd0[k] = zero16
        xd2[k] = zero16
        return 0

    lax.fori_loop(0, KA, zb, 0)
    rows0 = sid * ROWS_PT
    for z in range(ROWS_PT // KA):
        pltpu.sync_copy(hb0, accN.at[pl.ds(rows0 + KA * z, KA), :])
        pltpu.sync_copy(xd0, accD.at[pl.ds(rows0 + KA * z, KA), :])
    plsc.subcore_barrier()

    def issue(c, p):
        sidx = tsrc.at[c]
        pltpu.async_copy(g1_hbm.at[sidx], hbufs[p], gsems[p])
        pltpu.async_copy(as_hbm.at[sidx], abufs[p], gsems[p])
        pltpu.async_copy(ad_hbm.at[tdst.at[c]], dbufs[p], gsems[p])

    def wait_gather(p):
        pltpu.make_async_copy(g1_hbm.at[tsrc.at[0]], hbufs[p], gsems[p]).wait()
        pltpu.make_async_copy(as_hbm.at[tsrc.at[0]], abufs[p], gsems[p]).wait()
        pltpu.make_async_copy(ad_hbm.at[tdst.at[0]], dbufs[p], gsems[p]).wait()

    def issue_scatter(c, p):
        pltpu.async_copy(hbufs[p], accN.at[tdst.at[c]], ssems[p], add=True)
        pltpu.async_copy(xbufs[p], accD.at[tdst.at[c]], ssems[p], add=True)

    def wait_scatter(p):
        pltpu.make_async_copy(hbufs[p], accN.at[tdst.at[0]], ssems[p]).wait()
        pltpu.make_async_copy(xbufs[p], accD.at[tdst.at[0]], ssems[p]).wait()

    def compute(p):
        hb, ab, db, xd = hbufs[p], abufs[p], dbufs[p], xbufs[p]

        def body(k, _):
            e = ab[k] + db[k]
            e = jnp.maximum(e, 0.2 * e)
            ev = jnp.exp(e - CC)
            xd[k] = ev
            for hh in range(8):
                s = ev[hh]
                hb[k, pl.ds(16 * hh, 16)] = hb[k, pl.ds(16 * hh, 16)] * s
            return 0

        lax.fori_loop(0, KA, body, 0)

    issue(0, 0)
    issue(1, 1)
    issue_scatter(0, 2)

    def outer(g, _):
        for b in range(3):
            c = 3 * g + b
            p = b
            r = (b + 2) % 3
            wait_gather(p)
            compute(p)
            issue_scatter(c, p)
            wait_scatter(r)
            issue(c + 2, r)
        return 0

    lax.fori_loop(0, CHA // 3, outer, 0)
    wait_scatter(2)
    wait_gather(0)
    wait_gather(1)
    plsc.subcore_barrier()
    pltpu.sync_copy(accN.at[pl.ds(rows0, ROWS_PT), :],
                    accn_hbm.at[cid, pl.ds(rows0, ROWS_PT), :])
    pltpu.sync_copy(accD.at[pl.ds(rows0, ROWS_PT), :],
                    accd_hbm.at[cid, pl.ds(rows0, ROWS_PT), :])


def _sc_agg2(g2_hbm, a2_hbm, src1_hbm, dst1_hbm, m2_hbm,
             accn_hbm, accd_hbm,
             tsf, tdf, ta, tb, ccb,
             gb0, gb1, gb2, gb3, xb0, xb1, xb2, xb3,
             accN, accD,
             gs0, gs1, gs2, gs3, ss0, ss1, ss2, ss3):
    cid = lax.axis_index("c")
    sid = lax.axis_index("s")
    w = sid * 2 + cid
    pltpu.sync_copy(src1_hbm.at[pl.ds(w * EPT, EPT)], tsf.at[pl.ds(0, EPT)])
    pltpu.sync_copy(dst1_hbm.at[pl.ds(w * EPT, EPT)], tdf.at[pl.ds(0, EPT)])
    pltpu.sync_copy(src1_hbm.at[pl.ds(w * EPT, 2 * K)],
                    tsf.at[pl.ds(EPT, 2 * K)])
    pltpu.sync_copy(dst1_hbm.at[pl.ds(w * EPT, 2 * K)],
                    tdf.at[pl.ds(EPT, 2 * K)])
    pltpu.sync_copy(a2_hbm.at[pl.ds(0, NPAD)], ta)
    pltpu.sync_copy(a2_hbm.at[pl.ds(NPAD, NPAD)], tb)
    pltpu.sync_copy(m2_hbm, ccb)
    mv = ccb[0]
    t2 = mv[0] + mv[8]
    c2 = jnp.maximum(t2, 0.2 * t2)

    gbufs = (gb0, gb1, gb2, gb3)
    xbufs = (xb0, xb1, xb2, xb3)
    gsems = (gs0, gs1, gs2, gs3)
    ssems = (ss0, ss1, ss2, ss3)

    zero16 = jnp.zeros((16,), jnp.float32)

    def zb(k, _):
        gb0[k] = zero16
        gb2[k] = zero16
        gb3[k] = zero16
        xb0[k] = zero16
        xb2[k] = zero16
        xb3[k] = zero16
        return 0

    lax.fori_loop(0, K, zb, 0)
    rows0 = sid * ROWS_PT
    off = 0
    for nrow in _ZCHUNKS:
        pltpu.sync_copy(gb0.at[pl.ds(0, nrow), :],
                        accN.at[pl.ds(rows0 + off, nrow), :])
        pltpu.sync_copy(xb0.at[pl.ds(0, nrow), :],
                        accD.at[pl.ds(rows0 + off, nrow), :])
        off += nrow
    plsc.subcore_barrier()

    def issue(c, p):
        pltpu.async_copy(g2_hbm.at[tsf.at[pl.ds(c * K, K)]], gbufs[p],
                         gsems[p])

    def wait_gather(p):
        pltpu.make_async_copy(g2_hbm.at[tsf.at[pl.ds(0, K)]], gbufs[p],
                              gsems[p]).wait()

    def issue_scatter(c, p):
        didx = tdf.at[pl.ds(c * K, K)]
        pltpu.async_copy(gbufs[p], accN.at[didx], ssems[p], add=True)
        pltpu.async_copy(xbufs[p], accD.at[didx], ssems[p], add=True)

    def wait_scatter(p):
        didx = tdf.at[pl.ds(0, K)]
        pltpu.make_async_copy(gbufs[p], accN.at[didx], ssems[p]).wait()
        pltpu.make_async_copy(xbufs[p], accD.at[didx], ssems[p]).wait()

    def compute(c, p):
        gb, xb = gbufs[p], xbufs[p]

        def body(j, _):
            o = c * K + j * 16
            sv = tsf[pl.ds(o, 16)]
            dv = tdf[pl.ds(o, 16)]
            e = plsc.load_gather(ta, [sv]) + plsc.load_gather(tb, [dv])
            e = jnp.maximum(e, 0.2 * e)
            ev = jnp.exp(e - c2)
            for l in range(16):
                s = ev[l]
                k = j * 16 + l
                gb[k] = gb[k] * s
                xb[k] = jnp.full((16,), 1.0, jnp.float32) * s
            return 0

        lax.fori_loop(0, K // 16, body, 0)

    issue(0, 0)
    issue(1, 1)
    issue_scatter(0, 2)
    issue_scatter(0, 3)

    def outer(g, _):
        for b in range(NBUF):
            c = 4 * g + b
            p = b
            q = (b + 2) % NBUF
            wait_gather(p)
            compute(c, p)
            issue_scatter(c, p)
            wait_scatter(q)
            issue(c + 2, q)
        return 0

    lax.fori_loop(0, CH // 4, outer, 0)
    wait_scatter(2)
    wait_scatter(3)
    wait_gather(0)
    wait_gather(1)
    plsc.subcore_barrier()
    pltpu.sync_copy(accN.at[pl.ds(rows0, ROWS_PT), :],
                    accn_hbm.at[cid, pl.ds(rows0, ROWS_PT), :])
    pltpu.sync_copy(accD.at[pl.ds(rows0, ROWS_PT), :],
                    accd_hbm.at[cid, pl.ds(rows0, ROWS_PT), :])


def _make_sc_kernels():
    mesh = plsc.VectorSubcoreMesh(**_MESH)
    f32, i32 = jnp.float32, jnp.int32
    sca = functools.partial(
        pl.kernel, _sc_agg1,
        out_type=(jax.ShapeDtypeStruct((2, NPAD, 128), f32),
                  jax.ShapeDtypeStruct((2, NPAD, 16), f32)),
        mesh=mesh,
        compiler_params=_SC_PARAMS,
        scratch_types=(
            [pltpu.VMEM((CHA + 2, KA), i32), pltpu.VMEM((CHA + 2, KA), i32),
             pltpu.VMEM((1, 16), f32)]
            + [pltpu.VMEM((KA, 128), f32)] * 3
            + [pltpu.VMEM((KA, 16), f32)] * 9
            + [pltpu.VMEM_SHARED((NPAD, 128), f32),
               pltpu.VMEM_SHARED((NPAD, 16), f32)]
            + [pltpu.SemaphoreType.DMA] * 6
        ))()
    scb = functools.partial(
        pl.kernel, _sc_agg2,
        out_type=(jax.ShapeDtypeStruct((2, NPAD, 16), f32),
                  jax.ShapeDtypeStruct((2, NPAD, 16), f32)),
        mesh=mesh,
        compiler_params=_SC_PARAMS,
        scratch_types=(
            [pltpu.VMEM((EPT + 2 * K,), i32), pltpu.VMEM((EPT + 2 * K,), i32),
             pltpu.VMEM((NPAD,), f32), pltpu.VMEM((NPAD,), f32),
             pltpu.VMEM((1, 16), f32)]
            + [pltpu.VMEM((K, 16), f32)] * 8
            + [pltpu.VMEM_SHARED((NPAD, 16), f32),
               pltpu.VMEM_SHARED((NPAD, 16), f32)]
            + [pltpu.SemaphoreType.DMA] * 8
        ))()
    return sca, scb


_SCA, _SCB = _make_sc_kernels()


def kernel(x, edge_index, W1, a_src1, a_dst1, b1, W2, a_src2, a_dst2, b2):
    f32 = jnp.float32
    loop = jnp.arange(N, dtype=jnp.int32)
    pad = jnp.full((EPAD - E_REAL,), N, jnp.int32)
    src = jnp.concatenate([edge_index[0].astype(jnp.int32), loop, pad])
    dst = jnp.concatenate([edge_index[1].astype(jnp.int32), loop, pad])
    src3a = src.reshape(NW, CHA, KA)
    dst3a = dst.reshape(NW, CHA, KA)
    x_pad = jnp.zeros((NPAD, 128), f32).at[:N].set(x)

    BLK = 1024
    grid = NPAD // BLK
    g1, asd, add_, ms, md = pl.pallas_call(
        _tc_prep1,
        grid=(grid,),
        in_specs=[
            pl.BlockSpec((BLK, 128), lambda i: (i, 0)),
            pl.BlockSpec((128, 128), lambda i: (0, 0)),
            pl.BlockSpec((1, 128), lambda i: (0, 0)),
            pl.BlockSpec((1, 128), lambda i: (0, 0)),
        ],
        out_specs=[
            pl.BlockSpec((BLK, 128), lambda i: (i, 0)),
            pl.BlockSpec((BLK, 16), lambda i: (i, 0)),
            pl.BlockSpec((BLK, 16), lambda i: (i, 0)),
            pl.BlockSpec((1, 16), lambda i: (0, 0)),
            pl.BlockSpec((1, 16), lambda i: (0, 0)),
        ],
        out_shape=[
            jax.ShapeDtypeStruct((NPAD, 128), f32),
            jax.ShapeDtypeStruct((NPAD, 16), f32),
            jax.ShapeDtypeStruct((NPAD, 16), f32),
            jax.ShapeDtypeStruct((1, 16), f32),
            jax.ShapeDtypeStruct((1, 16), f32),
        ],
    )(x_pad, W1, a_src1.reshape(1, 128), a_dst1.reshape(1, 128))

    accn, accd = _SCA(g1, asd, add_, src3a, dst3a, ms, md)

    g2, a2, m2 = pl.pallas_call(
        _tc_mid,
        grid=(grid,),
        in_specs=[
            pl.BlockSpec((2, BLK, 128), lambda i: (0, i, 0)),
            pl.BlockSpec((2, BLK, 16), lambda i: (0, i, 0)),
            pl.BlockSpec((128,), lambda i: (0,)),
            pl.BlockSpec((128, 16), lambda i: (0, 0)),
            pl.BlockSpec((1, 16), lambda i: (0, 0)),
            pl.BlockSpec((1, 16), lambda i: (0, 0)),
        ],
        out_specs=[
            pl.BlockSpec((BLK, 16), lambda i: (i, 0)),
            pl.BlockSpec((8, BLK), lambda i: (0, i)),
            pl.BlockSpec((1, 16), lambda i: (0, 0)),
        ],
        out_shape=[
            jax.ShapeDtypeStruct((NPAD, 16), f32),
            jax.ShapeDtypeStruct((8, NPAD), f32),
            jax.ShapeDtypeStruct((1, 16), f32),
        ],
    )(accn, accd, b1, W2, a_src2, a_dst2)

    accn2, accd2 = _SCB(g2, a2.reshape(8 * NPAD), src, dst, m2)

    BLK2 = 1000
    out = pl.pallas_call(
        _tc_final,
        grid=(N // BLK2,),
        in_specs=[
            pl.BlockSpec((2, BLK2, 16), lambda i: (0, i, 0)),
            pl.BlockSpec((2, BLK2, 16), lambda i: (0, i, 0)),
            pl.BlockSpec((16,), lambda i: (0,)),
        ],
        out_specs=pl.BlockSpec((BLK2, 16), lambda i: (i, 0)),
        out_shape=jax.ShapeDtypeStruct((N, 16), f32),
    )(accn2[:, :N], accd2[:, :N], b2)
    return out


# speedup vs baseline: 90.7542x; 1.0123x over previous
import functools

import jax
import jax.numpy as jnp
from jax import lax
from jax.experimental import pallas as pl
from jax.experimental.pallas import tpu as pltpu
from jax.experimental.pallas import tpu_sc as plsc

N = 10000
NPAD = 10240
E_REAL = 330000
NW = 32
EPT = 10368
EPAD = NW * EPT
CH = 108
K = 96
CHA = 324
KA = 32
ROWS_PT = NPAD // 16
NBUF = 4
_ZCHUNKS = (96, 96, 96, 96, 96, 96, 64)

_MESH = dict(core_axis_name="c", subcore_axis_name="s", num_cores=2,
             num_subcores=16)
_SC_PARAMS = pltpu.CompilerParams(needs_layout_passes=False,
                                  use_tc_tiling_on_sc=False)


def _tc_prep1(x_ref, w_ref, asr_ref, adr_ref, g_ref, as_ref, ad_ref,
              ms_ref, md_ref):
    h = jnp.dot(x_ref[...], w_ref[...], preferred_element_type=jnp.float32)
    g_ref[...] = h
    sel = (lax.broadcasted_iota(jnp.int32, (128, 8), 0) // 16
           == lax.broadcasted_iota(jnp.int32, (128, 8), 1)).astype(jnp.float32)
    asr = jnp.dot(h * asr_ref[...], sel, preferred_element_type=jnp.float32)
    adr = jnp.dot(h * adr_ref[...], sel, preferred_element_type=jnp.float32)
    as_ref[...] = jnp.concatenate([asr, asr], axis=1)
    ad_ref[...] = jnp.concatenate([adr, adr], axis=1)
    bs = jnp.max(asr, axis=0, keepdims=True)
    bd = jnp.max(adr, axis=0, keepdims=True)
    bs = jnp.concatenate([bs, bs], axis=1)
    bd = jnp.concatenate([bd, bd], axis=1)
    i = pl.program_id(0)
    big = jnp.full((1, 16), -3.0e38, jnp.float32)
    ms_ref[...] = jnp.maximum(jnp.where(i == 0, big, ms_ref[...]), bs)
    md_ref[...] = jnp.maximum(jnp.where(i == 0, big, md_ref[...]), bd)


def _tc_mid(an_ref, ad_ref, b1_ref, w2_ref, s2_ref, d2_ref,
            g2_ref, a2_ref, m2_ref):
    num = an_ref[0] + an_ref[1]
    den = ad_ref[0] + ad_ref[1]
    den128 = jnp.repeat(den[:, 0:8], 16, axis=1)
    o = num / (den128 + 1e-16) + b1_ref[...][None, :]
    h2 = jnp.where(o > 0, o, jnp.exp(o) - 1.0)
    g = jnp.dot(h2, w2_ref[...], preferred_element_type=jnp.float32)
    g2_ref[...] = g
    s2 = jnp.dot(g, s2_ref[...].T, preferred_element_type=jnp.float32)
    d2 = jnp.dot(g, d2_ref[...].T, preferred_element_type=jnp.float32)
    B = s2.shape[0]
    a2_ref[...] = jnp.concatenate(
        [s2.T, d2.T, jnp.zeros((6, B), jnp.float32)], axis=0)
    bs = jnp.broadcast_to(jnp.max(s2, axis=0, keepdims=True), (1, 8))
    bd = jnp.broadcast_to(jnp.max(d2, axis=0, keepdims=True), (1, 8))
    m2 = jnp.concatenate([bs, bd], axis=1)
    i = pl.program_id(0)
    big = jnp.full((1, 16), -3.0e38, jnp.float32)
    m2_ref[...] = jnp.maximum(jnp.where(i == 0, big, m2_ref[...]), m2)


def _tc_final(an_ref, ad_ref, b2_ref, o_ref):
    num = an_ref[0] + an_ref[1]
    den = ad_ref[0] + ad_ref[1]
    o = num / (den[:, 0:1] + 1e-16) + b2_ref[...][None, :]
    m = jnp.max(o, axis=1, keepdims=True)
    s = jnp.log(jnp.sum(jnp.exp(o - m), axis=1, keepdims=True))
    o_ref[...] = o - m - s


def _sc_agg1(g1_hbm, as_hbm, ad_hbm, src_hbm, dst_hbm, ms_hbm, md_hbm,
             accn_hbm, accd_hbm,
             tsrc, tdst, ccb,
             hb0, hb1, hb2, ab0, ab1, ab2,
             db0, db1, db2, xd0, xd1, xd2,
             accN, accD,
             gs0, gs1, gs2, ss0, ss1, ss2):
    cid = lax.axis_index("c")
    sid = lax.axis_index("s")
    w = sid * 2 + cid
    pltpu.async_copy(src_hbm.at[w], tsrc.at[pl.ds(0, CHA), :], gs0)
    pltpu.async_copy(dst_hbm.at[w], tdst.at[pl.ds(0, CHA), :], gs1)
    pltpu.async_copy(src_hbm.at[w, pl.ds(0, 2), :], tsrc.at[pl.ds(CHA, 2), :],
                     gs2)
    pltpu.async_copy(dst_hbm.at[w, pl.ds(0, 2), :], tdst.at[pl.ds(CHA, 2), :],
                     gs2)
    pltpu.sync_copy(ms_hbm, ccb)
    msv = ccb[0]
    pltpu.sync_copy(md_hbm, ccb)
    t = msv + ccb[0]
    CC = jnp.maximum(t, 0.2 * t)

    hbufs = (hb0, hb1, hb2)
    abufs = (ab0, ab1, ab2)
    dbufs = (db0, db1, db2)
    xbufs = (xd0, xd1, xd2)
    gsems = (gs0, gs1, gs2)
    ssems = (ss0, ss1, ss2)

    zero16 = jnp.zeros((16,), jnp.float32)
    rows0 = sid * ROWS_PT

    def zb(k, _):
        for hh in range(8):
            hb0[k, pl.ds(16 * hh, 16)] = zero16
            hb2[k, pl.ds(16 * hh, 16)] = zero16
        xd0[k] = zero16
        xd2[k] = zero16
        return 0

    lax.fori_loop(0, KA, zb, 0)
    for z in range(ROWS_PT // KA):
        pltpu.async_copy(hb0, accN.at[pl.ds(rows0 + KA * z, KA), :], ss0)
        pltpu.async_copy(xd0, accD.at[pl.ds(rows0 + KA * z, KA), :], ss1)
    for z in range(ROWS_PT // KA):
        pltpu.make_async_copy(hb0, accN.at[pl.ds(rows0, KA), :], ss0).wait()
        pltpu.make_async_copy(xd0, accD.at[pl.ds(rows0, KA), :], ss1).wait()
    pltpu.make_async_copy(src_hbm.at[w], tsrc.at[pl.ds(0, CHA), :],
                          gs0).wait()
    pltpu.make_async_copy(dst_hbm.at[w], tdst.at[pl.ds(0, CHA), :],
                          gs1).wait()
    pltpu.make_async_copy(src_hbm.at[w, pl.ds(0, 2), :],
                          tsrc.at[pl.ds(CHA, 2), :], gs2).wait()
    pltpu.make_async_copy(src_hbm.at[w, pl.ds(0, 2), :],
                          tsrc.at[pl.ds(CHA, 2), :], gs2).wait()
    plsc.subcore_barrier()

    def issue(c, p):
        sidx = tsrc.at[c]
        pltpu.async_copy(g1_hbm.at[sidx], hbufs[p], gsems[p])
        pltpu.async_copy(as_hbm.at[sidx], abufs[p], gsems[p])
        pltpu.async_copy(ad_hbm.at[tdst.at[c]], dbufs[p], gsems[p])

    def wait_gather(p):
        pltpu.make_async_copy(g1_hbm.at[tsrc.at[0]], hbufs[p], gsems[p]).wait()
        pltpu.make_async_copy(as_hbm.at[tsrc.at[0]], abufs[p], gsems[p]).wait()
        pltpu.make_async_copy(ad_hbm.at[tdst.at[0]], dbufs[p], gsems[p]).wait()

    def issue_scatter(c, p):
        pltpu.async_copy(hbufs[p], accN.at[tdst.at[c]], ssems[p], add=True)
        pltpu.async_copy(xbufs[p], accD.at[tdst.at[c]], ssems[p], add=True)

    def wait_scatter(p):
        pltpu.make_async_copy(hbufs[p], accN.at[tdst.at[0]], ssems[p]).wait()
        pltpu.make_async_copy(xbufs[p], accD.at[tdst.at[0]], ssems[p]).wait()

    def compute(p):
        hb, ab, db, xd = hbufs[p], abufs[p], dbufs[p], xbufs[p]

        def body(j, _):
            for u in range(4):
                k = 4 * j + u
                e = ab[k] + db[k]
                e = jnp.maximum(e, 0.2 * e)
                ev = jnp.exp(e - CC)
                xd[k] = ev
                for hh in range(8):
                    s = ev[hh]
                    hb[k, pl.ds(16 * hh, 16)] = hb[k, pl.ds(16 * hh, 16)] * s
            return 0

        lax.fori_loop(0, KA // 4, body, 0)

    issue(0, 0)
    issue(1, 1)
    issue_scatter(0, 2)

    def outer(g, _):
        for b in range(3):
            c = 3 * g + b
            p = b
            r = (b + 2) % 3
            wait_gather(p)
            compute(p)
            issue_scatter(c, p)
            wait_scatter(r)
            issue(c + 2, r)
        return 0

    lax.fori_loop(0, CHA // 3, outer, 0)
    wait_scatter(2)
    wait_gather(0)
    wait_gather(1)
    plsc.subcore_barrier()
    pltpu.sync_copy(accN.at[pl.ds(rows0, ROWS_PT), :],
                    accn_hbm.at[cid, pl.ds(rows0, ROWS_PT), :])
    pltpu.sync_copy(accD.at[pl.ds(rows0, ROWS_PT), :],
                    accd_hbm.at[cid, pl.ds(rows0, ROWS_PT), :])


def _sc_agg2(g2_hbm, a2_hbm, src1_hbm, dst1_hbm, m2_hbm,
             accn_hbm, accd_hbm,
             tsf, tdf, ta, tb, ccb,
             gb0, gb1, gb2, gb3, xb0, xb1, xb2, xb3,
             accN, accD,
             gs0, gs1, gs2, gs3, ss0, ss1, ss2, ss3):
    cid = lax.axis_index("c")
    sid = lax.axis_index("s")
    w = sid * 2 + cid
    pltpu.async_copy(src1_hbm.at[pl.ds(w * EPT, EPT)], tsf.at[pl.ds(0, EPT)],
                     gs0)
    pltpu.async_copy(dst1_hbm.at[pl.ds(w * EPT, EPT)], tdf.at[pl.ds(0, EPT)],
                     gs1)
    pltpu.async_copy(src1_hbm.at[pl.ds(w * EPT, 2 * K)],
                     tsf.at[pl.ds(EPT, 2 * K)], gs2)
    pltpu.async_copy(dst1_hbm.at[pl.ds(w * EPT, 2 * K)],
                     tdf.at[pl.ds(EPT, 2 * K)], gs2)
    pltpu.async_copy(a2_hbm.at[pl.ds(0, NPAD)], ta, gs3)
    pltpu.async_copy(a2_hbm.at[pl.ds(NPAD, NPAD)], tb, gs3)
    pltpu.sync_copy(m2_hbm, ccb)
    mv = ccb[0]
    t2 = mv[0] + mv[8]
    c2 = jnp.maximum(t2, 0.2 * t2)

    gbufs = (gb0, gb1, gb2, gb3)
    xbufs = (xb0, xb1, xb2, xb3)
    gsems = (gs0, gs1, gs2, gs3)
    ssems = (ss0, ss1, ss2, ss3)

    zero16 = jnp.zeros((16,), jnp.float32)
    rows0 = sid * ROWS_PT

    def zb(k, _):
        gb0[k] = zero16
        gb2[k] = zero16
        gb3[k] = zero16
        xb0[k] = zero16
        xb2[k] = zero16
        xb3[k] = zero16
        return 0

    lax.fori_loop(0, K, zb, 0)
    off = 0
    for nrow in _ZCHUNKS:
        pltpu.async_copy(gb0.at[pl.ds(0, nrow), :],
                         accN.at[pl.ds(rows0 + off, nrow), :], ss0)
        pltpu.async_copy(xb0.at[pl.ds(0, nrow), :],
                         accD.at[pl.ds(rows0 + off, nrow), :], ss1)
        off += nrow
    off = 0
    for nrow in _ZCHUNKS:
        pltpu.make_async_copy(gb0.at[pl.ds(0, nrow), :],
                              accN.at[pl.ds(rows0 + off, nrow), :],
                              ss0).wait()
        pltpu.make_async_copy(xb0.at[pl.ds(0, nrow), :],
                              accD.at[pl.ds(rows0 + off, nrow), :],
                              ss1).wait()
        off += nrow
    pltpu.make_async_copy(src1_hbm.at[pl.ds(w * EPT, EPT)],
                          tsf.at[pl.ds(0, EPT)], gs0).wait()
    pltpu.make_async_copy(dst1_hbm.at[pl.ds(w * EPT, EPT)],
                          tdf.at[pl.ds(0, EPT)], gs1).wait()
    pltpu.make_async_copy(src1_hbm.at[pl.ds(w * EPT, 2 * K)],
                          tsf.at[pl.ds(EPT, 2 * K)], gs2).wait()
    pltpu.make_async_copy(src1_hbm.at[pl.ds(w * EPT, 2 * K)],
                          tsf.at[pl.ds(EPT, 2 * K)], gs2).wait()
    pltpu.make_async_copy(a2_hbm.at[pl.ds(0, NPAD)], ta, gs3).wait()
    pltpu.make_async_copy(a2_hbm.at[pl.ds(0, NPAD)], ta, gs3).wait()
    plsc.subcore_barrier()

    def issue(c, p):
        pltpu.async_copy(g2_hbm.at[tsf.at[pl.ds(c * K, K)]], gbufs[p],
                         gsems[p])

    def wait_gather(p):
        pltpu.make_async_copy(g2_hbm.at[tsf.at[pl.ds(0, K)]], gbufs[p],
                              gsems[p]).wait()

    def issue_scatter(c, p):
        didx = tdf.at[pl.ds(c * K, K)]
        pltpu.async_copy(gbufs[p], accN.at[didx], ssems[p], add=True)
        pltpu.async_copy(xbufs[p], accD.at[didx], ssems[p], add=True)

    def wait_scatter(p):
        didx = tdf.at[pl.ds(0, K)]
        pltpu.make_async_copy(gbufs[p], accN.at[didx], ssems[p]).wait()
        pltpu.make_async_copy(xbufs[p], accD.at[didx], ssems[p]).wait()

    def compute(c, p):
        gb, xb = gbufs[p], xbufs[p]

        def body(j, _):
            o = c * K + j * 16
            sv = tsf[pl.ds(o, 16)]
            dv = tdf[pl.ds(o, 16)]
            e = plsc.load_gather(ta, [sv]) + plsc.load_gather(tb, [dv])
            e = jnp.maximum(e, 0.2 * e)
            ev = jnp.exp(e - c2)
            for l in range(16):
                s = ev[l]
                k = j * 16 + l
                gb[k] = gb[k] * s
                xb[k] = jnp.full((16,), 1.0, jnp.float32) * s
            return 0

        lax.fori_loop(0, K // 16, body, 0)

    issue(0, 0)
    issue(1, 1)
    issue_scatter(0, 2)
    issue_scatter(0, 3)

    def outer(g, _):
        for b in range(NBUF):
            c = 4 * g + b
            p = b
            q = (b + 2) % NBUF
            wait_gather(p)
            compute(c, p)
            issue_scatter(c, p)
            wait_scatter(q)
            issue(c + 2, q)
        return 0

    lax.fori_loop(0, CH // 4, outer, 0)
    wait_scatter(2)
    wait_scatter(3)
    wait_gather(0)
    wait_gather(1)
    plsc.subcore_barrier()
    pltpu.sync_copy(accN.at[pl.ds(rows0, ROWS_PT), :],
                    accn_hbm.at[cid, pl.ds(rows0, ROWS_PT), :])
    pltpu.sync_copy(accD.at[pl.ds(rows0, ROWS_PT), :],
                    accd_hbm.at[cid, pl.ds(rows0, ROWS_PT), :])


def _make_sc_kernels():
    mesh = plsc.VectorSubcoreMesh(**_MESH)
    f32, i32 = jnp.float32, jnp.int32
    sca = functools.partial(
        pl.kernel, _sc_agg1,
        out_type=(jax.ShapeDtypeStruct((2, NPAD, 128), f32),
                  jax.ShapeDtypeStruct((2, NPAD, 16), f32)),
        mesh=mesh,
        compiler_params=_SC_PARAMS,
        scratch_types=(
            [pltpu.VMEM((CHA + 2, KA), i32), pltpu.VMEM((CHA + 2, KA), i32),
             pltpu.VMEM((1, 16), f32)]
            + [pltpu.VMEM((KA, 128), f32)] * 3
            + [pltpu.VMEM((KA, 16), f32)] * 9
            + [pltpu.VMEM_SHARED((NPAD, 128), f32),
               pltpu.VMEM_SHARED((NPAD, 16), f32)]
            + [pltpu.SemaphoreType.DMA] * 6
        ))()
    scb = functools.partial(
        pl.kernel, _sc_agg2,
        out_type=(jax.ShapeDtypeStruct((2, NPAD, 16), f32),
                  jax.ShapeDtypeStruct((2, NPAD, 16), f32)),
        mesh=mesh,
        compiler_params=_SC_PARAMS,
        scratch_types=(
            [pltpu.VMEM((EPT + 2 * K,), i32), pltpu.VMEM((EPT + 2 * K,), i32),
             pltpu.VMEM((NPAD,), f32), pltpu.VMEM((NPAD,), f32),
             pltpu.VMEM((1, 16), f32)]
            + [pltpu.VMEM((K, 16), f32)] * 8
            + [pltpu.VMEM_SHARED((NPAD, 16), f32),
               pltpu.VMEM_SHARED((NPAD, 16), f32)]
            + [pltpu.SemaphoreType.DMA] * 8
        ))()
    return sca, scb


_SCA, _SCB = _make_sc_kernels()


def kernel(x, edge_index, W1, a_src1, a_dst1, b1, W2, a_src2, a_dst2, b2):
    f32 = jnp.float32
    loop = jnp.arange(N, dtype=jnp.int32)
    pad = jnp.full((EPAD - E_REAL,), N, jnp.int32)
    src = jnp.concatenate([edge_index[0].astype(jnp.int32), loop, pad])
    dst = jnp.concatenate([edge_index[1].astype(jnp.int32), loop, pad])
    src3a = src.reshape(NW, CHA, KA)
    dst3a = dst.reshape(NW, CHA, KA)
    x_pad = jnp.zeros((NPAD, 128), f32).at[:N].set(x)

    BLK = 1024
    grid = NPAD // BLK
    g1, asd, add_, ms, md = pl.pallas_call(
        _tc_prep1,
        grid=(grid,),
        in_specs=[
            pl.BlockSpec((BLK, 128), lambda i: (i, 0)),
            pl.BlockSpec((128, 128), lambda i: (0, 0)),
            pl.BlockSpec((1, 128), lambda i: (0, 0)),
            pl.BlockSpec((1, 128), lambda i: (0, 0)),
        ],
        out_specs=[
            pl.BlockSpec((BLK, 128), lambda i: (i, 0)),
            pl.BlockSpec((BLK, 16), lambda i: (i, 0)),
            pl.BlockSpec((BLK, 16), lambda i: (i, 0)),
            pl.BlockSpec((1, 16), lambda i: (0, 0)),
            pl.BlockSpec((1, 16), lambda i: (0, 0)),
        ],
        out_shape=[
            jax.ShapeDtypeStruct((NPAD, 128), f32),
            jax.ShapeDtypeStruct((NPAD, 16), f32),
            jax.ShapeDtypeStruct((NPAD, 16), f32),
            jax.ShapeDtypeStruct((1, 16), f32),
            jax.ShapeDtypeStruct((1, 16), f32),
        ],
    )(x_pad, W1, a_src1.reshape(1, 128), a_dst1.reshape(1, 128))

    accn, accd = _SCA(g1, asd, add_, src3a, dst3a, ms, md)

    g2, a2, m2 = pl.pallas_call(
        _tc_mid,
        grid=(grid,),
        in_specs=[
            pl.BlockSpec((2, BLK, 128), lambda i: (0, i, 0)),
            pl.BlockSpec((2, BLK, 16), lambda i: (0, i, 0)),
            pl.BlockSpec((128,), lambda i: (0,)),
            pl.BlockSpec((128, 16), lambda i: (0, 0)),
            pl.BlockSpec((1, 16), lambda i: (0, 0)),
            pl.BlockSpec((1, 16), lambda i: (0, 0)),
        ],
        out_specs=[
            pl.BlockSpec((BLK, 16), lambda i: (i, 0)),
            pl.BlockSpec((8, BLK), lambda i: (0, i)),
            pl.BlockSpec((1, 16), lambda i: (0, 0)),
        ],
        out_shape=[
            jax.ShapeDtypeStruct((NPAD, 16), f32),
            jax.ShapeDtypeStruct((8, NPAD), f32),
            jax.ShapeDtypeStruct((1, 16), f32),
        ],
    )(accn, accd, b1, W2, a_src2, a_dst2)

    accn2, accd2 = _SCB(g2, a2.reshape(8 * NPAD), src, dst, m2)

    BLK2 = 1000
    out = pl.pallas_call(
        _tc_final,
        grid=(N // BLK2,),
        in_specs=[
            pl.BlockSpec((2, BLK2, 16), lambda i: (0, i, 0)),
            pl.BlockSpec((2, BLK2, 16), lambda i: (0, i, 0)),
            pl.BlockSpec((16,), lambda i: (0,)),
        ],
        out_specs=pl.BlockSpec((BLK2, 16), lambda i: (i, 0)),
        out_shape=jax.ShapeDtypeStruct((N, 16), f32),
    )(accn2[:, :N], accd2[:, :N], b2)
    return out
